# baseline (speedup 1.0000x reference)
import jax
import jax.numpy as jnp
from jax.experimental import pallas as pl
from jax.experimental.pallas import tpu as pltpu

_F = 10
_PF = 16
_ONES = 10


def _mlp_body(x_ref, a_ref, c_ref, b_ref, o_ref):
    xb = x_ref[...]
    h = jnp.dot(xb, a_ref[...], preferred_element_type=jnp.float32)
    h = jnp.maximum(h + c_ref[...], 0.0)
    y = jnp.dot(h, b_ref[...], preferred_element_type=jnp.float32)
    o_ref[...] = y[:, :_F].astype(o_ref.dtype)


def kernel(x, packed_params):
    B = x.shape[0]
    f32 = jnp.float32
    p = packed_params.astype(f32)

    a = p[0, :, :_F].T
    c = p[0, :, _ONES][None]
    b2 = p[1].T

    tm = 8192
    b_pad = -(-B // tm) * tm
    xin = x.astype(f32)
    if b_pad != B:
        xin = jnp.pad(xin, ((0, b_pad - B), (0, 0)))

    y = pl.pallas_call(
        _mlp_body,
        out_shape=jax.ShapeDtypeStruct((b_pad, _F), f32),
        grid=(b_pad // tm,),
        in_specs=[
            pl.BlockSpec((tm, _F), lambda i: (i, 0)),
            pl.BlockSpec((_F, _PF), lambda i: (0, 0)),
            pl.BlockSpec((1, _PF), lambda i: (0, 0)),
            pl.BlockSpec((_PF, _PF), lambda i: (0, 0)),
        ],
        out_specs=pl.BlockSpec((tm, _F), lambda i: (i, 0)),
        compiler_params=pltpu.CompilerParams(
            dimension_semantics=("parallel",)),
    )(xin, a, c, b2)

    return y[:B]
